# baseline (speedup 1.0000x reference)
import functools

import jax
import jax.numpy as jnp
from jax import lax
from jax.experimental import pallas as pl
from jax.experimental.pallas import tpu as pltpu
from jax.experimental.pallas import tpu_sc as plsc

N = 10000
E = 320000
D = 128
H = 256
NC = 2
NS = 16
C = 80
BN = 2000
NB = N // BN
EPS = 1e-5

_mesh = plsc.VectorSubcoreMesh(core_axis_name="c", subcore_axis_name="s")


def _make_agg(fh, rows_pt, split_edges):
    SB = 8
    K = 3
    nsb = rows_pt // SB
    rem = rows_pt - nsb * SB

    @functools.partial(
        pl.kernel,
        out_type=jax.ShapeDtypeStruct((2 * N, fh), jnp.float32),
        mesh=_mesh,
        scratch_types=[
            pltpu.VMEM_SHARED((N, fh), jnp.float32),
            pltpu.VMEM((SB, C), jnp.int32),
            pltpu.VMEM((SB, C), jnp.int32),
        ] + [pltpu.VMEM((C, fh), jnp.float32) for _ in range(K)] + [
            pltpu.SemaphoreType.DMA,
            pltpu.SemaphoreType.DMA,
        ],
    )
    def agg(src_hbm, dst_hbm, u_hbm, z_hbm, s_hbm, acc, sidx, didx,
            b0, b1, b2, gsem, ssem):
        bufs = [b0, b1, b2]
        c = lax.axis_index("c")
        w = lax.axis_index("s")

        @pl.when(w < 10)
        def _zero():
            pltpu.sync_copy(z_hbm, acc.at[pl.ds(w * 1000, 1000)])

        plsc.subcore_barrier()

        def gather(k, buf):
            return pltpu.async_copy(u_hbm.at[sidx.at[k]], buf, gsem)

        def scat(k, buf):
            return pltpu.async_copy(buf, acc.at[didx.at[k]], ssem, add=True)

        def run_block(nrows):
            gd = [None] * nrows
            sd = [None] * nrows
            for k in range(min(K, nrows)):
                gd[k] = gather(k, bufs[k])
            for k in range(nrows):
                gd[k].wait()
                sd[k] = scat(k, bufs[k % K])
                if k + K < nrows:
                    sd[k].wait()
                    gd[k + K] = gather(k + K, bufs[k % K])
            for k in range(max(nrows - K, 0), nrows):
                sd[k].wait()

        def load_idx(row, n):
            pltpu.sync_copy(src_hbm.at[c, w, pl.ds(row, n)],
                            sidx.at[pl.ds(0, n)])
            if split_edges:
                pltpu.sync_copy(dst_hbm.at[c, w, pl.ds(row, n)],
                                didx.at[pl.ds(0, n)])
            else:
                pltpu.sync_copy(dst_hbm.at[w, pl.ds(row, n)],
                                didx.at[pl.ds(0, n)])

        def body(j, carry):
            load_idx(j * SB, SB)
            run_block(SB)
            return carry

        lax.fori_loop(0, nsb, body, 0)
        if rem:
            load_idx(nsb * SB, rem)
            run_block(rem)
        plsc.subcore_barrier()

        @pl.when(w < 10)
        def _wb():
            pltpu.sync_copy(acc.at[pl.ds(w * 1000, 1000)],
                            s_hbm.at[pl.ds(c * N + w * 1000, 1000)])

    return agg


_agg128 = _make_agg(128, E // NS // C, False)
_agg0 = _make_agg(D, E // NC // NS // C, True)


@functools.partial(
    pl.kernel,
    out_type=jax.ShapeDtypeStruct((2 * N,), jnp.float32),
    mesh=_mesh,
    scratch_types=[
        pltpu.VMEM_SHARED((N,), jnp.float32),
        pltpu.VMEM((125, C), jnp.int32),
        pltpu.VMEM((C,), jnp.float32),
        pltpu.VMEM((1000,), jnp.float32),
        pltpu.SemaphoreType.DMA,
    ],
)
def _deg(dst_hbm, ones_hbm, z_hbm, out_hbm, acc, didx, ones_v, stage, ssem):
    c = lax.axis_index("c")
    w = lax.axis_index("s")
    pltpu.sync_copy(dst_hbm.at[c, w], didx)
    pltpu.sync_copy(ones_hbm, ones_v)

    @pl.when(w < 10)
    def _zero():
        pltpu.sync_copy(z_hbm, stage)
        pltpu.sync_copy(stage, acc.at[pl.ds(w * 1000, 1000)])

    plsc.subcore_barrier()

    def fire(i, carry):
        pltpu.async_copy(ones_v, acc.at[didx.at[i]], ssem, add=True)
        return carry

    lax.fori_loop(0, 125, fire, 0)

    def drain(i, carry):
        pltpu.make_async_copy(ones_v, acc.at[didx.at[i]], ssem).wait()
        return carry

    lax.fori_loop(0, 125, drain, 0)
    plsc.subcore_barrier()

    @pl.when(w < 10)
    def _wb():
        pltpu.sync_copy(acc.at[pl.ds(w * 1000, 1000)], stage)
        pltpu.sync_copy(stage, out_hbm.at[pl.ds(c * N + w * 1000, 1000)])


@functools.partial(
    pl.kernel,
    out_type=jax.ShapeDtypeStruct((2 * N,), jnp.float32),
    mesh=_mesh,
    scratch_types=[
        pltpu.VMEM_SHARED((N,), jnp.float32),
        pltpu.VMEM((125, C), jnp.int32),
        pltpu.VMEM((125, C), jnp.int32),
        pltpu.VMEM((125, C), jnp.float32),
        pltpu.VMEM((1000,), jnp.float32),
        pltpu.SemaphoreType.DMA,
        pltpu.SemaphoreType.DMA,
    ],
)
def _agg1(src_hbm, dst_hbm, u_hbm, z_hbm, out_hbm, acc, sidx, didx, vals,
          stage, sem_g, sem_s):
    c = lax.axis_index("c")
    w = lax.axis_index("s")
    pltpu.sync_copy(src_hbm.at[c, w], sidx)
    pltpu.sync_copy(dst_hbm.at[c, w], didx)

    @pl.when(w < 10)
    def _zero():
        pltpu.sync_copy(z_hbm, stage)
        pltpu.sync_copy(stage, acc.at[pl.ds(w * 1000, 1000)])

    plsc.subcore_barrier()

    def fire_g(i, carry):
        pltpu.async_copy(u_hbm.at[sidx.at[i]], vals.at[i], sem_g)
        return carry

    lax.fori_loop(0, 125, fire_g, 0)

    def fire_s(i, carry):
        pltpu.make_async_copy(u_hbm.at[sidx.at[i]], vals.at[i], sem_g).wait()
        pltpu.async_copy(vals.at[i], acc.at[didx.at[i]], sem_s, add=True)
        return carry

    lax.fori_loop(0, 125, fire_s, 0)

    def drain(i, carry):
        pltpu.make_async_copy(vals.at[i], acc.at[didx.at[i]], sem_s).wait()
        return carry

    lax.fori_loop(0, 125, drain, 0)
    plsc.subcore_barrier()

    @pl.when(w < 10)
    def _wb():
        pltpu.sync_copy(acc.at[pl.ds(w * 1000, 1000)], stage)
        pltpu.sync_copy(stage, out_hbm.at[pl.ds(c * N + w * 1000, 1000)])


def _p0_body(dega_ref, degb_ref, x_ref, d_ref, u_ref):
    dv = lax.rsqrt(1.0 + dega_ref[...] + degb_ref[...])
    d_ref[...] = dv
    u_ref[...] = x_ref[...] * dv


def _p0(dega, degb, x):
    return pl.pallas_call(
        _p0_body,
        grid=(NB,),
        in_specs=[
            pl.BlockSpec((BN, 1), lambda i: (i, 0)),
            pl.BlockSpec((BN, 1), lambda i: (i, 0)),
            pl.BlockSpec((BN, D), lambda i: (i, 0)),
        ],
        out_specs=[
            pl.BlockSpec((BN, 1), lambda i: (i, 0)),
            pl.BlockSpec((BN, D), lambda i: (i, 0)),
        ],
        out_shape=[
            jax.ShapeDtypeStruct((N, 1), jnp.float32),
            jax.ShapeDtypeStruct((N, D), jnp.float32),
        ],
    )(dega, degb, x)


def _make_layer(l0, with_t):
    def body(s_ref, u_ref, d_ref, w_ref, b_ref, g_ref, be_ref, *refs):
        if with_t:
            w3_ref = refs[0]
            refs = refs[1:]
        uo_ref = refs[0]
        t_ref = refs[1] if with_t else None
        zscr, ssum, s2sum = refs[-3:]
        ph = pl.program_id(0)
        i = pl.program_id(1)

        @pl.when(ph == 0)
        def _compute():
            dv = d_ref[...]
            if l0:
                y = (s_ref[0] + s_ref[1] + u_ref[...]) * dv
                z = jnp.dot(y, w_ref[...], preferred_element_type=jnp.float32,
                            precision=lax.Precision.HIGHEST) + b_ref[...]
            else:
                y0 = (s_ref[0] + u_ref[0]) * dv
                y1 = (s_ref[1] + u_ref[1]) * dv
                z = (jnp.dot(y0, w_ref[:128],
                             preferred_element_type=jnp.float32,
                             precision=lax.Precision.HIGHEST)
                     + jnp.dot(y1, w_ref[128:],
                               preferred_element_type=jnp.float32,
                               precision=lax.Precision.HIGHEST)
                     + b_ref[...])
            zscr[pl.ds(i * BN, BN), :] = z

            @pl.when(i == 0)
            def _init():
                ssum[...] = jnp.zeros_like(ssum)
                s2sum[...] = jnp.zeros_like(s2sum)

            ssum[...] += jnp.sum(z, axis=0, keepdims=True)
            s2sum[...] += jnp.sum(z * z, axis=0, keepdims=True)

        @pl.when(ph == 1)
        def _apply():
            z = zscr[pl.ds(i * BN, BN), :]
            m = ssum[...] / N
            v = s2sum[...] / N - m * m
            inv = lax.rsqrt(v + EPS) * g_ref[...]
            h = jnp.maximum((z - m) * inv + be_ref[...], 0.0)
            uu = h * d_ref[...]
            uo_ref[0] = uu[:, :128]
            uo_ref[1] = uu[:, 128:]
            if with_t:
                t_ref[...] = jnp.sum(uu * w3_ref[...], axis=1, keepdims=True)

    fh = D if l0 else 128
    s_spec = pl.BlockSpec((2, BN, fh), lambda p, i: (0, i, 0))
    u_spec = (pl.BlockSpec((BN, D), lambda p, i: (i, 0)) if l0
              else pl.BlockSpec((2, BN, 128), lambda p, i: (0, i, 0)))
    in_specs = [
        s_spec,
        u_spec,
        pl.BlockSpec((BN, 1), lambda p, i: (i, 0)),
        pl.BlockSpec((D if l0 else H, H), lambda p, i: (0, 0)),
        pl.BlockSpec((1, H), lambda p, i: (0, 0)),
        pl.BlockSpec((1, H), lambda p, i: (0, 0)),
        pl.BlockSpec((1, H), lambda p, i: (0, 0)),
    ]
    out_specs = [pl.BlockSpec((2, BN, 128), lambda p, i: (0, i, 0))]
    out_shape = [jax.ShapeDtypeStruct((2, N, 128), jnp.float32)]
    if with_t:
        in_specs.append(pl.BlockSpec((1, H), lambda p, i: (0, 0)))
        out_specs.append(pl.BlockSpec((BN, 1), lambda p, i: (i, 0)))
        out_shape.append(jax.ShapeDtypeStruct((N, 1), jnp.float32))

    def run(*args):
        return pl.pallas_call(
            body,
            grid=(2, NB),
            in_specs=in_specs,
            out_specs=out_specs,
            out_shape=out_shape,
            scratch_shapes=[
                pltpu.VMEM((N, H), jnp.float32),
                pltpu.VMEM((1, H), jnp.float32),
                pltpu.VMEM((1, H), jnp.float32),
            ],
        )(*args)

    return run


_layer0 = _make_layer(True, False)
_layer1 = _make_layer(False, False)
_layer2 = _make_layer(False, True)


def _p4_body(sa_ref, sb_ref, u_ref, d_ref, b3_ref, o_ref):
    o_ref[...] = (d_ref[...] * (sa_ref[...] + sb_ref[...] + u_ref[...])
                  + b3_ref[...])


def _p4(sa, sb, u3, d, b3):
    return pl.pallas_call(
        _p4_body,
        in_specs=[
            pl.BlockSpec((N, 1), lambda: (0, 0)),
            pl.BlockSpec((N, 1), lambda: (0, 0)),
            pl.BlockSpec((N, 1), lambda: (0, 0)),
            pl.BlockSpec((N, 1), lambda: (0, 0)),
            pl.BlockSpec((1, 1), lambda: (0, 0)),
        ],
        out_specs=pl.BlockSpec((N, 1), lambda: (0, 0)),
        out_shape=jax.ShapeDtypeStruct((N, 1), jnp.float32),
    )(sa, sb, u3, d, b3)


def kernel(x, edge_index, W0, b0, g0, be0, W1, b1, g1, be1,
           W2, b2, g2, be2, W3, b3):
    src = edge_index[0]
    dst = edge_index[1]
    src_off = jnp.stack([src, src + N]).reshape(2, NS, E // (NS * C), C)
    src2 = src.reshape(2, NS, E // (2 * NS * C), C)
    dst2 = dst.reshape(2, NS, E // (2 * NS * C), C)
    dst_all = dst.reshape(NS, E // (NS * C), C)
    zeros128 = jnp.zeros((1000, 128), jnp.float32)
    zeros1 = jnp.zeros((1000,), jnp.float32)
    ones_c = jnp.ones((C,), jnp.float32)

    degs = _deg(dst2, ones_c, zeros1)
    d, u0 = _p0(degs[:N].reshape(N, 1), degs[N:].reshape(N, 1), x)

    s0 = _agg0(src2, dst2, u0, zeros128)
    u1 = _layer0(s0.reshape(2, N, D), u0, d, W0, b0.reshape(1, H),
                 g0.reshape(1, H), be0.reshape(1, H))[0]

    s1 = _agg128(src_off, dst_all, u1.reshape(2 * N, 128), zeros128)
    u2 = _layer1(s1.reshape(2, N, 128), u1, d, W1, b1.reshape(1, H),
                 g1.reshape(1, H), be1.reshape(1, H))[0]

    s2a = _agg128(src_off, dst_all, u2.reshape(2 * N, 128), zeros128)
    _, t = _layer2(s2a.reshape(2, N, 128), u2, d, W2, b2.reshape(1, H),
                   g2.reshape(1, H), be2.reshape(1, H), W3.reshape(1, H))
    s3 = _agg1(src2, dst2, t.reshape(N), zeros1)
    out = _p4(s3[:N].reshape(N, 1), s3[N:].reshape(N, 1), t, d,
              b3.reshape(1, 1))
    return out.reshape(N)


# speedup vs baseline: 1.1447x; 1.1447x over previous
import functools

import jax
import jax.numpy as jnp
from jax import lax
from jax.experimental import pallas as pl
from jax.experimental.pallas import tpu as pltpu
from jax.experimental.pallas import tpu_sc as plsc

N = 10000
E = 320000
D = 128
H = 256
NC = 2
NS = 16
C = 80
BN = 2000
NB = N // BN
EPS = 1e-5

_mesh = plsc.VectorSubcoreMesh(core_axis_name="c", subcore_axis_name="s")


def _make_agg(fh, rows_pt, split_edges):
    SB = 32
    K = 3
    nsb = rows_pt // SB
    rem = rows_pt - nsb * SB

    @functools.partial(
        pl.kernel,
        out_type=jax.ShapeDtypeStruct((2 * N, fh), jnp.float32),
        mesh=_mesh,
        scratch_types=[
            pltpu.VMEM_SHARED((N, fh), jnp.float32),
            pltpu.VMEM((SB, C), jnp.int32),
            pltpu.VMEM((SB, C), jnp.int32),
        ] + [pltpu.VMEM((C, fh), jnp.float32) for _ in range(K)] + [
            pltpu.SemaphoreType.DMA,
            pltpu.SemaphoreType.DMA,
        ],
    )
    def agg(src_hbm, dst_hbm, u_hbm, z_hbm, s_hbm, acc, sidx, didx,
            b0, b1, b2, gsem, ssem):
        bufs = [b0, b1, b2]
        c = lax.axis_index("c")
        w = lax.axis_index("s")

        @pl.when(w < 10)
        def _zero():
            pltpu.sync_copy(z_hbm, acc.at[pl.ds(w * 1000, 1000)])

        plsc.subcore_barrier()

        def gather(k, buf):
            return pltpu.async_copy(u_hbm.at[sidx.at[k]], buf, gsem)

        def scat(k, buf):
            return pltpu.async_copy(buf, acc.at[didx.at[k]], ssem, add=True)

        def run_block(nrows):
            gd = [None] * nrows
            sd = [None] * nrows
            for k in range(min(K, nrows)):
                gd[k] = gather(k, bufs[k])
            for k in range(nrows):
                gd[k].wait()
                sd[k] = scat(k, bufs[k % K])
                if k + K < nrows:
                    sd[k].wait()
                    gd[k + K] = gather(k + K, bufs[k % K])
            for k in range(max(nrows - K, 0), nrows):
                sd[k].wait()

        def load_idx(row, n):
            pltpu.sync_copy(src_hbm.at[c, w, pl.ds(row, n)],
                            sidx.at[pl.ds(0, n)])
            if split_edges:
                pltpu.sync_copy(dst_hbm.at[c, w, pl.ds(row, n)],
                                didx.at[pl.ds(0, n)])
            else:
                pltpu.sync_copy(dst_hbm.at[w, pl.ds(row, n)],
                                didx.at[pl.ds(0, n)])

        def body(j, carry):
            load_idx(j * SB, SB)
            run_block(SB)
            return carry

        lax.fori_loop(0, nsb, body, 0)
        if rem:
            load_idx(nsb * SB, rem)
            run_block(rem)
        plsc.subcore_barrier()

        @pl.when(w < 10)
        def _wb():
            pltpu.sync_copy(acc.at[pl.ds(w * 1000, 1000)],
                            s_hbm.at[pl.ds(c * N + w * 1000, 1000)])

    return agg


_agg128 = _make_agg(128, E // NS // C, False)
_agg0 = _make_agg(D, E // NC // NS // C, True)


@functools.partial(
    pl.kernel,
    out_type=jax.ShapeDtypeStruct((2 * N,), jnp.float32),
    mesh=_mesh,
    scratch_types=[
        pltpu.VMEM_SHARED((N,), jnp.float32),
        pltpu.VMEM((125, C), jnp.int32),
        pltpu.VMEM((C,), jnp.float32),
        pltpu.VMEM((1000,), jnp.float32),
        pltpu.SemaphoreType.DMA,
    ],
)
def _deg(dst_hbm, ones_hbm, z_hbm, out_hbm, acc, didx, ones_v, stage, ssem):
    c = lax.axis_index("c")
    w = lax.axis_index("s")
    pltpu.sync_copy(dst_hbm.at[c, w], didx)
    pltpu.sync_copy(ones_hbm, ones_v)

    @pl.when(w < 10)
    def _zero():
        pltpu.sync_copy(z_hbm, stage)
        pltpu.sync_copy(stage, acc.at[pl.ds(w * 1000, 1000)])

    plsc.subcore_barrier()

    def fire(i, carry):
        pltpu.async_copy(ones_v, acc.at[didx.at[i]], ssem, add=True)
        return carry

    lax.fori_loop(0, 125, fire, 0)

    def drain(i, carry):
        pltpu.make_async_copy(ones_v, acc.at[didx.at[i]], ssem).wait()
        return carry

    lax.fori_loop(0, 125, drain, 0)
    plsc.subcore_barrier()

    @pl.when(w < 10)
    def _wb():
        pltpu.sync_copy(acc.at[pl.ds(w * 1000, 1000)], stage)
        pltpu.sync_copy(stage, out_hbm.at[pl.ds(c * N + w * 1000, 1000)])


@functools.partial(
    pl.kernel,
    out_type=jax.ShapeDtypeStruct((2 * N,), jnp.float32),
    mesh=_mesh,
    scratch_types=[
        pltpu.VMEM_SHARED((N,), jnp.float32),
        pltpu.VMEM((125, C), jnp.int32),
        pltpu.VMEM((125, C), jnp.int32),
        pltpu.VMEM((125, C), jnp.float32),
        pltpu.VMEM((1000,), jnp.float32),
        pltpu.SemaphoreType.DMA,
        pltpu.SemaphoreType.DMA,
    ],
)
def _agg1(src_hbm, dst_hbm, u_hbm, z_hbm, out_hbm, acc, sidx, didx, vals,
          stage, sem_g, sem_s):
    c = lax.axis_index("c")
    w = lax.axis_index("s")
    pltpu.sync_copy(src_hbm.at[c, w], sidx)
    pltpu.sync_copy(dst_hbm.at[c, w], didx)

    @pl.when(w < 10)
    def _zero():
        pltpu.sync_copy(z_hbm, stage)
        pltpu.sync_copy(stage, acc.at[pl.ds(w * 1000, 1000)])

    plsc.subcore_barrier()

    def fire_g(i, carry):
        pltpu.async_copy(u_hbm.at[sidx.at[i]], vals.at[i], sem_g)
        return carry

    lax.fori_loop(0, 125, fire_g, 0)

    def fire_s(i, carry):
        pltpu.make_async_copy(u_hbm.at[sidx.at[i]], vals.at[i], sem_g).wait()
        pltpu.async_copy(vals.at[i], acc.at[didx.at[i]], sem_s, add=True)
        return carry

    lax.fori_loop(0, 125, fire_s, 0)

    def drain(i, carry):
        pltpu.make_async_copy(vals.at[i], acc.at[didx.at[i]], sem_s).wait()
        return carry

    lax.fori_loop(0, 125, drain, 0)
    plsc.subcore_barrier()

    @pl.when(w < 10)
    def _wb():
        pltpu.sync_copy(acc.at[pl.ds(w * 1000, 1000)], stage)
        pltpu.sync_copy(stage, out_hbm.at[pl.ds(c * N + w * 1000, 1000)])


def _p0_body(dega_ref, degb_ref, x_ref, d_ref, u_ref):
    dv = lax.rsqrt(1.0 + dega_ref[...] + degb_ref[...])
    d_ref[...] = dv
    u_ref[...] = x_ref[...] * dv


def _p0(dega, degb, x):
    return pl.pallas_call(
        _p0_body,
        grid=(NB,),
        in_specs=[
            pl.BlockSpec((BN, 1), lambda i: (i, 0)),
            pl.BlockSpec((BN, 1), lambda i: (i, 0)),
            pl.BlockSpec((BN, D), lambda i: (i, 0)),
        ],
        out_specs=[
            pl.BlockSpec((BN, 1), lambda i: (i, 0)),
            pl.BlockSpec((BN, D), lambda i: (i, 0)),
        ],
        out_shape=[
            jax.ShapeDtypeStruct((N, 1), jnp.float32),
            jax.ShapeDtypeStruct((N, D), jnp.float32),
        ],
    )(dega, degb, x)


def _make_layer(l0, with_t):
    def body(s_ref, u_ref, d_ref, w_ref, b_ref, g_ref, be_ref, *refs):
        if with_t:
            w3_ref = refs[0]
            refs = refs[1:]
        uo_ref = refs[0]
        t_ref = refs[1] if with_t else None
        zscr, ssum, s2sum = refs[-3:]
        ph = pl.program_id(0)
        i = pl.program_id(1)

        @pl.when(ph == 0)
        def _compute():
            dv = d_ref[...]
            if l0:
                y = (s_ref[0] + s_ref[1] + u_ref[...]) * dv
                z = jnp.dot(y, w_ref[...], preferred_element_type=jnp.float32,
                            precision=lax.Precision.HIGHEST) + b_ref[...]
            else:
                y0 = (s_ref[0] + u_ref[0]) * dv
                y1 = (s_ref[1] + u_ref[1]) * dv
                z = (jnp.dot(y0, w_ref[:128],
                             preferred_element_type=jnp.float32,
                             precision=lax.Precision.HIGHEST)
                     + jnp.dot(y1, w_ref[128:],
                               preferred_element_type=jnp.float32,
                               precision=lax.Precision.HIGHEST)
                     + b_ref[...])
            zscr[pl.ds(i * BN, BN), :] = z

            @pl.when(i == 0)
            def _init():
                ssum[...] = jnp.zeros_like(ssum)
                s2sum[...] = jnp.zeros_like(s2sum)

            ssum[...] += jnp.sum(z, axis=0, keepdims=True)
            s2sum[...] += jnp.sum(z * z, axis=0, keepdims=True)

        @pl.when(ph == 1)
        def _apply():
            z = zscr[pl.ds(i * BN, BN), :]
            m = ssum[...] / N
            v = s2sum[...] / N - m * m
            inv = lax.rsqrt(v + EPS) * g_ref[...]
            h = jnp.maximum((z - m) * inv + be_ref[...], 0.0)
            uu = h * d_ref[...]
            uo_ref[0] = uu[:, :128]
            uo_ref[1] = uu[:, 128:]
            if with_t:
                t_ref[...] = jnp.sum(uu * w3_ref[...], axis=1, keepdims=True)

    fh = D if l0 else 128
    s_spec = pl.BlockSpec((2, BN, fh), lambda p, i: (0, i, 0))
    u_spec = (pl.BlockSpec((BN, D), lambda p, i: (i, 0)) if l0
              else pl.BlockSpec((2, BN, 128), lambda p, i: (0, i, 0)))
    in_specs = [
        s_spec,
        u_spec,
        pl.BlockSpec((BN, 1), lambda p, i: (i, 0)),
        pl.BlockSpec((D if l0 else H, H), lambda p, i: (0, 0)),
        pl.BlockSpec((1, H), lambda p, i: (0, 0)),
        pl.BlockSpec((1, H), lambda p, i: (0, 0)),
        pl.BlockSpec((1, H), lambda p, i: (0, 0)),
    ]
    out_specs = [pl.BlockSpec((2, BN, 128), lambda p, i: (0, i, 0))]
    out_shape = [jax.ShapeDtypeStruct((2, N, 128), jnp.float32)]
    if with_t:
        in_specs.append(pl.BlockSpec((1, H), lambda p, i: (0, 0)))
        out_specs.append(pl.BlockSpec((BN, 1), lambda p, i: (i, 0)))
        out_shape.append(jax.ShapeDtypeStruct((N, 1), jnp.float32))

    def run(*args):
        return pl.pallas_call(
            body,
            grid=(2, NB),
            in_specs=in_specs,
            out_specs=out_specs,
            out_shape=out_shape,
            scratch_shapes=[
                pltpu.VMEM((N, H), jnp.float32),
                pltpu.VMEM((1, H), jnp.float32),
                pltpu.VMEM((1, H), jnp.float32),
            ],
        )(*args)

    return run


_layer0 = _make_layer(True, False)
_layer1 = _make_layer(False, False)
_layer2 = _make_layer(False, True)


def _p4_body(sa_ref, sb_ref, u_ref, d_ref, b3_ref, o_ref):
    o_ref[...] = (d_ref[...] * (sa_ref[...] + sb_ref[...] + u_ref[...])
                  + b3_ref[...])


def _p4(sa, sb, u3, d, b3):
    return pl.pallas_call(
        _p4_body,
        in_specs=[
            pl.BlockSpec((N, 1), lambda: (0, 0)),
            pl.BlockSpec((N, 1), lambda: (0, 0)),
            pl.BlockSpec((N, 1), lambda: (0, 0)),
            pl.BlockSpec((N, 1), lambda: (0, 0)),
            pl.BlockSpec((1, 1), lambda: (0, 0)),
        ],
        out_specs=pl.BlockSpec((N, 1), lambda: (0, 0)),
        out_shape=jax.ShapeDtypeStruct((N, 1), jnp.float32),
    )(sa, sb, u3, d, b3)


def kernel(x, edge_index, W0, b0, g0, be0, W1, b1, g1, be1,
           W2, b2, g2, be2, W3, b3):
    src = edge_index[0]
    dst = edge_index[1]
    src_off = jnp.stack([src, src + N]).reshape(2, NS, E // (NS * C), C)
    src2 = src.reshape(2, NS, E // (2 * NS * C), C)
    dst2 = dst.reshape(2, NS, E // (2 * NS * C), C)
    dst_all = dst.reshape(NS, E // (NS * C), C)
    zeros128 = jnp.zeros((1000, 128), jnp.float32)
    zeros1 = jnp.zeros((1000,), jnp.float32)
    ones_c = jnp.ones((C,), jnp.float32)

    degs = _deg(dst2, ones_c, zeros1)
    d, u0 = _p0(degs[:N].reshape(N, 1), degs[N:].reshape(N, 1), x)

    s0 = _agg0(src2, dst2, u0, zeros128)
    u1 = _layer0(s0.reshape(2, N, D), u0, d, W0, b0.reshape(1, H),
                 g0.reshape(1, H), be0.reshape(1, H))[0]

    s1 = _agg128(src_off, dst_all, u1.reshape(2 * N, 128), zeros128)
    u2 = _layer1(s1.reshape(2, N, 128), u1, d, W1, b1.reshape(1, H),
                 g1.reshape(1, H), be1.reshape(1, H))[0]

    s2a = _agg128(src_off, dst_all, u2.reshape(2 * N, 128), zeros128)
    _, t = _layer2(s2a.reshape(2, N, 128), u2, d, W2, b2.reshape(1, H),
                   g2.reshape(1, H), be2.reshape(1, H), W3.reshape(1, H))
    s3 = _agg1(src2, dst2, t.reshape(N), zeros1)
    out = _p4(s3[:N].reshape(N, 1), s3[N:].reshape(N, 1), t, d,
              b3.reshape(1, 1))
    return out.reshape(N)


# speedup vs baseline: 1.1504x; 1.0050x over previous
import functools

import jax
import jax.numpy as jnp
from jax import lax
from jax.experimental import pallas as pl
from jax.experimental.pallas import tpu as pltpu
from jax.experimental.pallas import tpu_sc as plsc

N = 10000
E = 320000
D = 128
H = 256
NC = 2
NS = 16
C = 80
BN = 2000
NB = N // BN
EPS = 1e-5

_mesh = plsc.VectorSubcoreMesh(core_axis_name="c", subcore_axis_name="s")


def _make_agg(fh, rows_pt, split_edges):
    SB = 40
    K = 3
    nsb = rows_pt // SB
    rem = rows_pt - nsb * SB

    @functools.partial(
        pl.kernel,
        out_type=jax.ShapeDtypeStruct((2 * N, fh), jnp.float32),
        mesh=_mesh,
        scratch_types=[
            pltpu.VMEM_SHARED((N, fh), jnp.float32),
            pltpu.VMEM((SB, C), jnp.int32),
            pltpu.VMEM((SB, C), jnp.int32),
        ] + [pltpu.VMEM((C, fh), jnp.float32) for _ in range(K)] + [
            pltpu.SemaphoreType.DMA,
            pltpu.SemaphoreType.DMA,
        ],
    )
    def agg(src_hbm, dst_hbm, u_hbm, z_hbm, s_hbm, acc, sidx, didx,
            b0, b1, b2, gsem, ssem):
        bufs = [b0, b1, b2]
        c = lax.axis_index("c")
        w = lax.axis_index("s")

        @pl.when(w < 10)
        def _zero():
            pltpu.sync_copy(z_hbm, acc.at[pl.ds(w * 1000, 1000)])

        plsc.subcore_barrier()

        def gather(k, buf):
            return pltpu.async_copy(u_hbm.at[sidx.at[k]], buf, gsem)

        def scat(k, buf):
            return pltpu.async_copy(buf, acc.at[didx.at[k]], ssem, add=True)

        def run_block(nrows):
            gd = [None] * nrows
            sd = [None] * nrows
            for k in range(min(K, nrows)):
                gd[k] = gather(k, bufs[k])
            for k in range(nrows):
                gd[k].wait()
                sd[k] = scat(k, bufs[k % K])
                if k + K < nrows:
                    sd[k].wait()
                    gd[k + K] = gather(k + K, bufs[k % K])
            for k in range(max(nrows - K, 0), nrows):
                sd[k].wait()

        def load_idx(row, n):
            pltpu.sync_copy(src_hbm.at[c, w, pl.ds(row, n)],
                            sidx.at[pl.ds(0, n)])
            if split_edges:
                pltpu.sync_copy(dst_hbm.at[c, w, pl.ds(row, n)],
                                didx.at[pl.ds(0, n)])
            else:
                pltpu.sync_copy(dst_hbm.at[w, pl.ds(row, n)],
                                didx.at[pl.ds(0, n)])

        def body(j, carry):
            load_idx(j * SB, SB)
            run_block(SB)
            return carry

        lax.fori_loop(0, nsb, body, 0)
        if rem:
            load_idx(nsb * SB, rem)
            run_block(rem)
        plsc.subcore_barrier()

        @pl.when(w < 10)
        def _wb():
            pltpu.sync_copy(acc.at[pl.ds(w * 1000, 1000)],
                            s_hbm.at[pl.ds(c * N + w * 1000, 1000)])

    return agg


_agg128 = _make_agg(128, E // NS // C, False)
_agg0 = _make_agg(D, E // NC // NS // C, True)


@functools.partial(
    pl.kernel,
    out_type=jax.ShapeDtypeStruct((2 * N,), jnp.float32),
    mesh=_mesh,
    scratch_types=[
        pltpu.VMEM_SHARED((N,), jnp.float32),
        pltpu.VMEM((125, C), jnp.int32),
        pltpu.VMEM((C,), jnp.float32),
        pltpu.VMEM((1000,), jnp.float32),
        pltpu.SemaphoreType.DMA,
    ],
)
def _deg(dst_hbm, ones_hbm, z_hbm, out_hbm, acc, didx, ones_v, stage, ssem):
    c = lax.axis_index("c")
    w = lax.axis_index("s")
    pltpu.sync_copy(dst_hbm.at[c, w], didx)
    pltpu.sync_copy(ones_hbm, ones_v)

    @pl.when(w < 10)
    def _zero():
        pltpu.sync_copy(z_hbm, stage)
        pltpu.sync_copy(stage, acc.at[pl.ds(w * 1000, 1000)])

    plsc.subcore_barrier()

    def fire(i, carry):
        pltpu.async_copy(ones_v, acc.at[didx.at[i]], ssem, add=True)
        return carry

    lax.fori_loop(0, 125, fire, 0)

    def drain(i, carry):
        pltpu.make_async_copy(ones_v, acc.at[didx.at[i]], ssem).wait()
        return carry

    lax.fori_loop(0, 125, drain, 0)
    plsc.subcore_barrier()

    @pl.when(w < 10)
    def _wb():
        pltpu.sync_copy(acc.at[pl.ds(w * 1000, 1000)], stage)
        pltpu.sync_copy(stage, out_hbm.at[pl.ds(c * N + w * 1000, 1000)])


@functools.partial(
    pl.kernel,
    out_type=jax.ShapeDtypeStruct((2 * N,), jnp.float32),
    mesh=_mesh,
    scratch_types=[
        pltpu.VMEM_SHARED((N,), jnp.float32),
        pltpu.VMEM((125, C), jnp.int32),
        pltpu.VMEM((125, C), jnp.int32),
        pltpu.VMEM((125, C), jnp.float32),
        pltpu.VMEM((1000,), jnp.float32),
        pltpu.SemaphoreType.DMA,
        pltpu.SemaphoreType.DMA,
    ],
)
def _agg1(src_hbm, dst_hbm, u_hbm, z_hbm, out_hbm, acc, sidx, didx, vals,
          stage, sem_g, sem_s):
    c = lax.axis_index("c")
    w = lax.axis_index("s")
    pltpu.sync_copy(src_hbm.at[c, w], sidx)
    pltpu.sync_copy(dst_hbm.at[c, w], didx)

    @pl.when(w < 10)
    def _zero():
        pltpu.sync_copy(z_hbm, stage)
        pltpu.sync_copy(stage, acc.at[pl.ds(w * 1000, 1000)])

    plsc.subcore_barrier()

    def fire_g(i, carry):
        pltpu.async_copy(u_hbm.at[sidx.at[i]], vals.at[i], sem_g)
        return carry

    lax.fori_loop(0, 125, fire_g, 0)

    def fire_s(i, carry):
        pltpu.make_async_copy(u_hbm.at[sidx.at[i]], vals.at[i], sem_g).wait()
        pltpu.async_copy(vals.at[i], acc.at[didx.at[i]], sem_s, add=True)
        return carry

    lax.fori_loop(0, 125, fire_s, 0)

    def drain(i, carry):
        pltpu.make_async_copy(vals.at[i], acc.at[didx.at[i]], sem_s).wait()
        return carry

    lax.fori_loop(0, 125, drain, 0)
    plsc.subcore_barrier()

    @pl.when(w < 10)
    def _wb():
        pltpu.sync_copy(acc.at[pl.ds(w * 1000, 1000)], stage)
        pltpu.sync_copy(stage, out_hbm.at[pl.ds(c * N + w * 1000, 1000)])


def _p0_body(dega_ref, degb_ref, x_ref, d_ref, u_ref):
    dv = lax.rsqrt(1.0 + dega_ref[...] + degb_ref[...])
    d_ref[...] = dv
    u_ref[...] = x_ref[...] * dv


def _p0(dega, degb, x):
    return pl.pallas_call(
        _p0_body,
        grid=(NB,),
        in_specs=[
            pl.BlockSpec((BN, 1), lambda i: (i, 0)),
            pl.BlockSpec((BN, 1), lambda i: (i, 0)),
            pl.BlockSpec((BN, D), lambda i: (i, 0)),
        ],
        out_specs=[
            pl.BlockSpec((BN, 1), lambda i: (i, 0)),
            pl.BlockSpec((BN, D), lambda i: (i, 0)),
        ],
        out_shape=[
            jax.ShapeDtypeStruct((N, 1), jnp.float32),
            jax.ShapeDtypeStruct((N, D), jnp.float32),
        ],
    )(dega, degb, x)


def _make_layer(l0, with_t):
    def body(s_ref, u_ref, d_ref, w_ref, b_ref, g_ref, be_ref, *refs):
        if with_t:
            w3_ref = refs[0]
            refs = refs[1:]
        uo_ref = refs[0]
        t_ref = refs[1] if with_t else None
        zscr, ssum, s2sum = refs[-3:]
        ph = pl.program_id(0)
        i = pl.program_id(1)

        @pl.when(ph == 0)
        def _compute():
            dv = d_ref[...]
            if l0:
                y = (s_ref[0] + s_ref[1] + u_ref[...]) * dv
                z = jnp.dot(y, w_ref[...], preferred_element_type=jnp.float32,
                            precision=lax.Precision.HIGHEST) + b_ref[...]
            else:
                y0 = (s_ref[0] + u_ref[0]) * dv
                y1 = (s_ref[1] + u_ref[1]) * dv
                z = (jnp.dot(y0, w_ref[:128],
                             preferred_element_type=jnp.float32,
                             precision=lax.Precision.HIGHEST)
                     + jnp.dot(y1, w_ref[128:],
                               preferred_element_type=jnp.float32,
                               precision=lax.Precision.HIGHEST)
                     + b_ref[...])
            zscr[pl.ds(i * BN, BN), :] = z

            @pl.when(i == 0)
            def _init():
                ssum[...] = jnp.zeros_like(ssum)
                s2sum[...] = jnp.zeros_like(s2sum)

            ssum[...] += jnp.sum(z, axis=0, keepdims=True)
            s2sum[...] += jnp.sum(z * z, axis=0, keepdims=True)

        @pl.when(ph == 1)
        def _apply():
            z = zscr[pl.ds(i * BN, BN), :]
            m = ssum[...] / N
            v = s2sum[...] / N - m * m
            inv = lax.rsqrt(v + EPS) * g_ref[...]
            h = jnp.maximum((z - m) * inv + be_ref[...], 0.0)
            uu = h * d_ref[...]
            uo_ref[0] = uu[:, :128]
            uo_ref[1] = uu[:, 128:]
            if with_t:
                t_ref[...] = jnp.sum(uu * w3_ref[...], axis=1, keepdims=True)

    fh = D if l0 else 128
    s_spec = pl.BlockSpec((2, BN, fh), lambda p, i: (0, i, 0))
    u_spec = (pl.BlockSpec((BN, D), lambda p, i: (i, 0)) if l0
              else pl.BlockSpec((2, BN, 128), lambda p, i: (0, i, 0)))
    in_specs = [
        s_spec,
        u_spec,
        pl.BlockSpec((BN, 1), lambda p, i: (i, 0)),
        pl.BlockSpec((D if l0 else H, H), lambda p, i: (0, 0)),
        pl.BlockSpec((1, H), lambda p, i: (0, 0)),
        pl.BlockSpec((1, H), lambda p, i: (0, 0)),
        pl.BlockSpec((1, H), lambda p, i: (0, 0)),
    ]
    out_specs = [pl.BlockSpec((2, BN, 128), lambda p, i: (0, i, 0))]
    out_shape = [jax.ShapeDtypeStruct((2, N, 128), jnp.float32)]
    if with_t:
        in_specs.append(pl.BlockSpec((1, H), lambda p, i: (0, 0)))
        out_specs.append(pl.BlockSpec((BN, 1), lambda p, i: (i, 0)))
        out_shape.append(jax.ShapeDtypeStruct((N, 1), jnp.float32))

    def run(*args):
        return pl.pallas_call(
            body,
            grid=(2, NB),
            in_specs=in_specs,
            out_specs=out_specs,
            out_shape=out_shape,
            scratch_shapes=[
                pltpu.VMEM((N, H), jnp.float32),
                pltpu.VMEM((1, H), jnp.float32),
                pltpu.VMEM((1, H), jnp.float32),
            ],
        )(*args)

    return run


_layer0 = _make_layer(True, False)
_layer1 = _make_layer(False, False)
_layer2 = _make_layer(False, True)


def _p4_body(sa_ref, sb_ref, u_ref, d_ref, b3_ref, o_ref):
    o_ref[...] = (d_ref[...] * (sa_ref[...] + sb_ref[...] + u_ref[...])
                  + b3_ref[...])


def _p4(sa, sb, u3, d, b3):
    return pl.pallas_call(
        _p4_body,
        in_specs=[
            pl.BlockSpec((N, 1), lambda: (0, 0)),
            pl.BlockSpec((N, 1), lambda: (0, 0)),
            pl.BlockSpec((N, 1), lambda: (0, 0)),
            pl.BlockSpec((N, 1), lambda: (0, 0)),
            pl.BlockSpec((1, 1), lambda: (0, 0)),
        ],
        out_specs=pl.BlockSpec((N, 1), lambda: (0, 0)),
        out_shape=jax.ShapeDtypeStruct((N, 1), jnp.float32),
    )(sa, sb, u3, d, b3)


def kernel(x, edge_index, W0, b0, g0, be0, W1, b1, g1, be1,
           W2, b2, g2, be2, W3, b3):
    src = edge_index[0]
    dst = edge_index[1]
    src_off = jnp.stack([src, src + N]).reshape(2, NS, E // (NS * C), C)
    src2 = src.reshape(2, NS, E // (2 * NS * C), C)
    dst2 = dst.reshape(2, NS, E // (2 * NS * C), C)
    dst_all = dst.reshape(NS, E // (NS * C), C)
    zeros128 = jnp.zeros((1000, 128), jnp.float32)
    zeros1 = jnp.zeros((1000,), jnp.float32)
    ones_c = jnp.ones((C,), jnp.float32)

    degs = _deg(dst2, ones_c, zeros1)
    d, u0 = _p0(degs[:N].reshape(N, 1), degs[N:].reshape(N, 1), x)

    s0 = _agg0(src2, dst2, u0, zeros128)
    u1 = _layer0(s0.reshape(2, N, D), u0, d, W0, b0.reshape(1, H),
                 g0.reshape(1, H), be0.reshape(1, H))[0]

    s1 = _agg128(src_off, dst_all, u1.reshape(2 * N, 128), zeros128)
    u2 = _layer1(s1.reshape(2, N, 128), u1, d, W1, b1.reshape(1, H),
                 g1.reshape(1, H), be1.reshape(1, H))[0]

    s2a = _agg128(src_off, dst_all, u2.reshape(2 * N, 128), zeros128)
    _, t = _layer2(s2a.reshape(2, N, 128), u2, d, W2, b2.reshape(1, H),
                   g2.reshape(1, H), be2.reshape(1, H), W3.reshape(1, H))
    s3 = _agg1(src2, dst2, t.reshape(N), zeros1)
    out = _p4(s3[:N].reshape(N, 1), s3[N:].reshape(N, 1), t, d,
              b3.reshape(1, 1))
    return out.reshape(N)
